# baseline (speedup 1.0000x reference)
import functools

import jax
import jax.numpy as jnp
from jax import lax
from jax.experimental import pallas as pl
from jax.experimental.pallas import tpu as pltpu
from jax.experimental.pallas import tpu_sc as plsc

N_NODES = 10000
N_EDGES = 320000

NC, NS = 2, 16
NW = NC * NS
CHUNK = 128
NCHUNK = 80
PER_W = NCHUNK * CHUNK
E_PAD = NW * PER_W
N_PAD = 10240
STRIPE = N_PAD // NS
ZR = 8
NZB = 4
NBUF = 2
IBLK = 16
NIB = NCHUNK // IBLK

BR = 2048
NBLK = 5


def _segsum_partials(src, dst, z, d):
    mesh = plsc.VectorSubcoreMesh(core_axis_name="c", subcore_axis_name="s")

    @functools.partial(
        pl.kernel,
        out_type=jax.ShapeDtypeStruct((NC * N_PAD, d), jnp.float32),
        mesh=mesh,
        scratch_types=[
            pltpu.VMEM((2, IBLK, CHUNK), jnp.int32),
            pltpu.VMEM((2, IBLK, CHUNK), jnp.int32),
            pltpu.VMEM((NBUF, CHUNK, d), jnp.float32),
            pltpu.VMEM((ZR, d), jnp.float32),
            pltpu.VMEM_SHARED((N_PAD, d), jnp.float32),
            pltpu.SemaphoreType.DMA((NBUF,)),
            pltpu.SemaphoreType.DMA((2, 2)),
            pltpu.SemaphoreType.DMA((NZB,)),
        ],
    )
    def seg_kernel(src_hbm, dst_hbm, z_hbm, out_hbm, sidx, didx, rows, zrows,
                   acc, gsem, isem, zsem):
        c = lax.axis_index("c")
        s = lax.axis_index("s")
        wid = c * NS + s
        base = wid * NCHUNK

        def idx_load(t, sl):
            rb = base + t * IBLK
            return (
                pltpu.make_async_copy(
                    src_hbm.at[pl.ds(rb, IBLK)], sidx.at[sl], isem.at[sl, 0]),
                pltpu.make_async_copy(
                    dst_hbm.at[pl.ds(rb, IBLK)], didx.at[sl], isem.at[sl, 1]),
            )

        def gather(b, sl, j):
            return pltpu.make_async_copy(
                z_hbm.at[sidx.at[sl, j]], rows.at[b], gsem.at[b])

        def scatter_add(b, sl, j):
            pltpu.sync_copy(rows.at[b], acc.at[didx.at[sl, j]], add=True)

        ld0 = idx_load(0, 0)
        for cp in ld0:
            cp.start()

        @pl.loop(0, ZR)
        def _(r):
            @pl.loop(0, d, step=16)
            def _(j):
                zrows.at[r, pl.ds(j, 16)][...] = jnp.zeros((16,), jnp.float32)

        NZC = STRIPE // ZR
        zcps = [
            pltpu.make_async_copy(
                zrows, acc.at[pl.ds(s * STRIPE + i * ZR, ZR)], zsem.at[i % NZB])
            for i in range(NZC)
        ]
        for i in range(NZC):
            if i >= NZB:
                zcps[i - NZB].wait()
            zcps[i].start()

        for cp in ld0:
            cp.wait()
        for b in range(NBUF):
            gather(b, 0, b).start()

        for i in range(NZC - NZB, NZC):
            zcps[i].wait()
        plsc.subcore_barrier()

        for t in range(NIB):
            sl = t % 2
            nsl = (t + 1) % 2
            if t < NIB - 1:
                nld = idx_load(t + 1, nsl)
                for cp in nld:
                    cp.start()

            @pl.loop(0, IBLK - NBUF, step=NBUF)
            def _(k):
                for b in range(NBUF):
                    j = k + b
                    gather(b, sl, j).wait()
                    scatter_add(b, sl, j)
                    gather(b, sl, j + NBUF).start()

            if t < NIB - 1:
                for cp in nld:
                    cp.wait()
                for b in range(NBUF):
                    j = IBLK - NBUF + b
                    gather(b, sl, j).wait()
                    scatter_add(b, sl, j)
                    gather(b, nsl, b).start()
            else:
                for b in range(NBUF):
                    j = IBLK - NBUF + b
                    gather(b, sl, j).wait()
                    scatter_add(b, sl, j)

        plsc.subcore_barrier()
        pltpu.sync_copy(
            acc.at[pl.ds(s * STRIPE, STRIPE)],
            out_hbm.at[pl.ds(c * N_PAD + s * STRIPE, STRIPE)],
        )

    return seg_kernel(src, dst, z)


def _matmul(x, wt):
    n, k = x.shape
    m = wt.shape[1]

    def body(x_ref, w_ref, o_ref):
        o_ref[...] = jnp.dot(x_ref[...], w_ref[...],
                             preferred_element_type=jnp.float32)

    return pl.pallas_call(
        body,
        grid=(NBLK,),
        in_specs=[
            pl.BlockSpec((BR, k), lambda i: (i, 0)),
            pl.BlockSpec((k, m), lambda i: (0, 0)),
        ],
        out_specs=pl.BlockSpec((BR, m), lambda i: (i, 0)),
        out_shape=jax.ShapeDtypeStruct((n, m), jnp.float32),
    )(x, wt)


def _combine_elu(g, p, b):
    n, k = g.shape
    poff = N_PAD // BR

    def body(g_ref, p0_ref, p1_ref, b_ref, o_ref):
        t = g_ref[...] + p0_ref[...] + p1_ref[...] + b_ref[...]
        o_ref[...] = jnp.where(t > 0, t, jnp.exp(t) - 1.0)

    return pl.pallas_call(
        body,
        grid=(NBLK,),
        in_specs=[
            pl.BlockSpec((BR, k), lambda i: (i, 0)),
            pl.BlockSpec((BR, k), lambda i: (i, 0)),
            pl.BlockSpec((BR, k), lambda i: (poff + i, 0)),
            pl.BlockSpec((1, k), lambda i: (0, 0)),
        ],
        out_specs=pl.BlockSpec((BR, k), lambda i: (i, 0)),
        out_shape=jax.ShapeDtypeStruct((n, k), jnp.float32),
    )(g, p, p, b.reshape(1, k))


def _combine_matmul_elu(h, p, wt, b):
    n, k = h.shape
    m = wt.shape[1]
    poff = N_PAD // BR

    def body(h_ref, p0_ref, p1_ref, w_ref, b_ref, o_ref):
        rst = h_ref[...] + p0_ref[...] + p1_ref[...]
        t = jnp.dot(rst, w_ref[...],
                    preferred_element_type=jnp.float32) + b_ref[...]
        o_ref[...] = jnp.where(t > 0, t, jnp.exp(t) - 1.0)

    return pl.pallas_call(
        body,
        grid=(NBLK,),
        in_specs=[
            pl.BlockSpec((BR, k), lambda i: (i, 0)),
            pl.BlockSpec((BR, k), lambda i: (i, 0)),
            pl.BlockSpec((BR, k), lambda i: (poff + i, 0)),
            pl.BlockSpec((k, m), lambda i: (0, 0)),
            pl.BlockSpec((1, m), lambda i: (0, 0)),
        ],
        out_specs=pl.BlockSpec((BR, m), lambda i: (i, 0)),
        out_shape=jax.ShapeDtypeStruct((n, m), jnp.float32),
    )(h, p, p, wt, b.reshape(1, m))


def kernel(features, edge_index, W1, b1, W2, b2):
    src = edge_index[0].astype(jnp.int32)
    dst = edge_index[1].astype(jnp.int32)
    npad = E_PAD - N_EDGES
    pad_i = jnp.arange(npad, dtype=jnp.int32)
    src_p = jnp.concatenate([src, pad_i % N_NODES]).reshape(E_PAD // CHUNK, CHUNK)
    dst_p = jnp.concatenate([dst, N_NODES + pad_i % (N_PAD - N_NODES)]).reshape(
        E_PAD // CHUNK, CHUNK)

    g1 = _matmul(features, W1.T)
    p1 = _segsum_partials(src_p, dst_p, g1, g1.shape[1])
    h = _combine_elu(g1, p1, b1)
    p2 = _segsum_partials(src_p, dst_p, h, h.shape[1])
    return _combine_matmul_elu(h, p2, W2.T, b2)


# speedup vs baseline: 1.0920x; 1.0920x over previous
import functools

import jax
import jax.numpy as jnp
from jax import lax
from jax.experimental import pallas as pl
from jax.experimental.pallas import tpu as pltpu
from jax.experimental.pallas import tpu_sc as plsc

N_NODES = 10000
N_EDGES = 320000

NC, NS = 2, 16
NW = NC * NS
CHUNK = 64
NCHUNK = 160
PER_W = NCHUNK * CHUNK
E_PAD = NW * PER_W
N_PAD = 10240
STRIPE = N_PAD // NS
ZR = 8
NZB = 4
NBUF = 4
IBLK = 16
NIB = NCHUNK // IBLK

BR = 2048
NBLK = 5


def _segsum_partials(src, dst, z, d):
    mesh = plsc.VectorSubcoreMesh(core_axis_name="c", subcore_axis_name="s")

    @functools.partial(
        pl.kernel,
        out_type=jax.ShapeDtypeStruct((NC * N_PAD, d), jnp.float32),
        mesh=mesh,
        scratch_types=[
            pltpu.VMEM((2, IBLK, CHUNK), jnp.int32),
            pltpu.VMEM((2, IBLK, CHUNK), jnp.int32),
            pltpu.VMEM((NBUF, CHUNK, d), jnp.float32),
            pltpu.VMEM((ZR, d), jnp.float32),
            pltpu.VMEM_SHARED((N_PAD, d), jnp.float32),
            pltpu.SemaphoreType.DMA((NBUF,)),
            pltpu.SemaphoreType.DMA((2, 2)),
            pltpu.SemaphoreType.DMA((NZB,)),
        ],
    )
    def seg_kernel(src_hbm, dst_hbm, z_hbm, out_hbm, sidx, didx, rows, zrows,
                   acc, gsem, isem, zsem):
        c = lax.axis_index("c")
        s = lax.axis_index("s")
        wid = c * NS + s
        base = wid * NCHUNK

        def idx_load(t, sl):
            rb = base + t * IBLK
            return (
                pltpu.make_async_copy(
                    src_hbm.at[pl.ds(rb, IBLK)], sidx.at[sl], isem.at[sl, 0]),
                pltpu.make_async_copy(
                    dst_hbm.at[pl.ds(rb, IBLK)], didx.at[sl], isem.at[sl, 1]),
            )

        def gather(b, sl, j):
            return pltpu.make_async_copy(
                z_hbm.at[sidx.at[sl, j]], rows.at[b], gsem.at[b])

        def scatter_add(b, sl, j):
            pltpu.sync_copy(rows.at[b], acc.at[didx.at[sl, j]], add=True)

        ld0 = idx_load(0, 0)
        for cp in ld0:
            cp.start()

        @pl.loop(0, ZR)
        def _(r):
            @pl.loop(0, d, step=16)
            def _(j):
                zrows.at[r, pl.ds(j, 16)][...] = jnp.zeros((16,), jnp.float32)

        NZC = STRIPE // ZR
        zcps = [
            pltpu.make_async_copy(
                zrows, acc.at[pl.ds(s * STRIPE + i * ZR, ZR)], zsem.at[i % NZB])
            for i in range(NZC)
        ]
        for i in range(NZC):
            if i >= NZB:
                zcps[i - NZB].wait()
            zcps[i].start()

        for cp in ld0:
            cp.wait()
        for b in range(NBUF):
            gather(b, 0, b).start()

        for i in range(NZC - NZB, NZC):
            zcps[i].wait()
        plsc.subcore_barrier()

        for t in range(NIB):
            sl = t % 2
            nsl = (t + 1) % 2
            if t < NIB - 1:
                nld = idx_load(t + 1, nsl)
                for cp in nld:
                    cp.start()

            @pl.loop(0, IBLK - NBUF, step=NBUF)
            def _(k):
                for b in range(NBUF):
                    j = k + b
                    gather(b, sl, j).wait()
                    scatter_add(b, sl, j)
                    gather(b, sl, j + NBUF).start()

            if t < NIB - 1:
                for cp in nld:
                    cp.wait()
                for b in range(NBUF):
                    j = IBLK - NBUF + b
                    gather(b, sl, j).wait()
                    scatter_add(b, sl, j)
                    gather(b, nsl, b).start()
            else:
                for b in range(NBUF):
                    j = IBLK - NBUF + b
                    gather(b, sl, j).wait()
                    scatter_add(b, sl, j)

        plsc.subcore_barrier()
        pltpu.sync_copy(
            acc.at[pl.ds(s * STRIPE, STRIPE)],
            out_hbm.at[pl.ds(c * N_PAD + s * STRIPE, STRIPE)],
        )

    return seg_kernel(src, dst, z)


def _matmul(x, wt):
    n, k = x.shape
    m = wt.shape[1]

    def body(x_ref, w_ref, o_ref):
        o_ref[...] = jnp.dot(x_ref[...], w_ref[...],
                             preferred_element_type=jnp.float32)

    return pl.pallas_call(
        body,
        grid=(NBLK,),
        in_specs=[
            pl.BlockSpec((BR, k), lambda i: (i, 0)),
            pl.BlockSpec((k, m), lambda i: (0, 0)),
        ],
        out_specs=pl.BlockSpec((BR, m), lambda i: (i, 0)),
        out_shape=jax.ShapeDtypeStruct((n, m), jnp.float32),
    )(x, wt)


def _combine_elu(g, p, b):
    n, k = g.shape
    poff = N_PAD // BR

    def body(g_ref, p0_ref, p1_ref, b_ref, o_ref):
        t = g_ref[...] + p0_ref[...] + p1_ref[...] + b_ref[...]
        o_ref[...] = jnp.where(t > 0, t, jnp.exp(t) - 1.0)

    return pl.pallas_call(
        body,
        grid=(NBLK,),
        in_specs=[
            pl.BlockSpec((BR, k), lambda i: (i, 0)),
            pl.BlockSpec((BR, k), lambda i: (i, 0)),
            pl.BlockSpec((BR, k), lambda i: (poff + i, 0)),
            pl.BlockSpec((1, k), lambda i: (0, 0)),
        ],
        out_specs=pl.BlockSpec((BR, k), lambda i: (i, 0)),
        out_shape=jax.ShapeDtypeStruct((n, k), jnp.float32),
    )(g, p, p, b.reshape(1, k))


def _combine_matmul_elu(h, p, wt, b):
    n, k = h.shape
    m = wt.shape[1]
    poff = N_PAD // BR

    def body(h_ref, p0_ref, p1_ref, w_ref, b_ref, o_ref):
        rst = h_ref[...] + p0_ref[...] + p1_ref[...]
        t = jnp.dot(rst, w_ref[...],
                    preferred_element_type=jnp.float32) + b_ref[...]
        o_ref[...] = jnp.where(t > 0, t, jnp.exp(t) - 1.0)

    return pl.pallas_call(
        body,
        grid=(NBLK,),
        in_specs=[
            pl.BlockSpec((BR, k), lambda i: (i, 0)),
            pl.BlockSpec((BR, k), lambda i: (i, 0)),
            pl.BlockSpec((BR, k), lambda i: (poff + i, 0)),
            pl.BlockSpec((k, m), lambda i: (0, 0)),
            pl.BlockSpec((1, m), lambda i: (0, 0)),
        ],
        out_specs=pl.BlockSpec((BR, m), lambda i: (i, 0)),
        out_shape=jax.ShapeDtypeStruct((n, m), jnp.float32),
    )(h, p, p, wt, b.reshape(1, m))


def kernel(features, edge_index, W1, b1, W2, b2):
    src = edge_index[0].astype(jnp.int32)
    dst = edge_index[1].astype(jnp.int32)
    npad = E_PAD - N_EDGES
    pad_i = jnp.arange(npad, dtype=jnp.int32)
    src_p = jnp.concatenate([src, pad_i % N_NODES]).reshape(E_PAD // CHUNK, CHUNK)
    dst_p = jnp.concatenate([dst, N_NODES + pad_i % (N_PAD - N_NODES)]).reshape(
        E_PAD // CHUNK, CHUNK)

    g1 = _matmul(features, W1.T)
    p1 = _segsum_partials(src_p, dst_p, g1, g1.shape[1])
    h = _combine_elu(g1, p1, b1)
    p2 = _segsum_partials(src_p, dst_p, h, h.shape[1])
    return _combine_matmul_elu(h, p2, W2.T, b2)
